# baseline (speedup 1.0000x reference)
import functools

import jax
import jax.numpy as jnp
from jax import lax
from jax.experimental import pallas as pl
from jax.experimental.pallas import tpu as pltpu
from jax.experimental.pallas import tpu_sc as plsc

ENT_TOTAL = 1000000
REL_TOTAL = 1000
HIDDEN = 16
BATCH = 16384
LMBDA = 0.1

NC = 2
NS = 16
NW = NC * NS
BPW = BATCH // NW
CHUNK = 64
NCHUNK = BPW // CHUNK
PACK = 128 // HIDDEN
ENT_PACKED = ENT_TOTAL // PACK
L = 16


def _sc_body(idh_hbm, idt_hbm, idr_hbm, emb_hbm, ere_hbm, eim_hbm,
             remb_hbm, rre_hbm, rim_hbm,
             res_hbm, reg_hbm,
             idh_v, idt_v, idr_v, ihi_v, iti_v,
             g_erh, g_eih, g_eh, g_ert, g_eit, g_et,
             remb_v, rre_v, rim_v,
             res_v, reg_v, sem):
  wid = lax.axis_index("s") * NC + lax.axis_index("c")
  base = wid * BPW
  row0 = wid * NCHUNK

  pltpu.sync_copy(idh_hbm.at[pl.ds(row0, NCHUNK)], idh_v)
  pltpu.sync_copy(idt_hbm.at[pl.ds(row0, NCHUNK)], idt_v)
  pltpu.sync_copy(idr_hbm.at[pl.ds(row0, NCHUNK)], idr_v)
  pltpu.sync_copy(remb_hbm, remb_v)
  pltpu.sync_copy(rre_hbm, rre_v)
  pltpu.sync_copy(rim_hbm, rim_v)

  for j in range(NCHUNK):
    for k in range(CHUNK // L):
      sl = pl.ds(k * L, L)
      ihi_v[j, sl] = idh_v[j, sl] >> 3
      iti_v[j, sl] = idt_v[j, sl] >> 3

  def chunk_body(j, accs):
    cps = [
        pltpu.async_copy(ere_hbm.at[ihi_v.at[j]], g_erh, sem),
        pltpu.async_copy(eim_hbm.at[ihi_v.at[j]], g_eih, sem),
        pltpu.async_copy(emb_hbm.at[ihi_v.at[j]], g_eh, sem),
        pltpu.async_copy(ere_hbm.at[iti_v.at[j]], g_ert, sem),
        pltpu.async_copy(eim_hbm.at[iti_v.at[j]], g_eit, sem),
        pltpu.async_copy(emb_hbm.at[iti_v.at[j]], g_et, sem),
    ]
    for cp in cps:
      cp.wait()

    accs = list(accs)
    for g in range(CHUNK // L):
      s = pl.ds(g * L, L)
      ids_h = idh_v[j, s]
      ids_t = idt_v[j, s]
      rid = idr_v[j, s]
      offh = (ids_h & (PACK - 1)) * HIDDEN
      offt = (ids_t & (PACK - 1)) * HIDDEN
      rowi = jnp.arange(L, dtype=jnp.int32) + (g * L)
      res_acc = jnp.zeros((L,), jnp.float32)
      for d in range(HIDDEN):
        ch = offh + d
        ct = offt + d
        cd = jnp.full((L,), d, jnp.int32)
        erh = plsc.load_gather(g_erh, [rowi, ch])
        eih = plsc.load_gather(g_eih, [rowi, ch])
        eh = plsc.load_gather(g_eh, [rowi, ch])
        ert = plsc.load_gather(g_ert, [rowi, ct])
        eit = plsc.load_gather(g_eit, [rowi, ct])
        et = plsc.load_gather(g_et, [rowi, ct])
        rre = plsc.load_gather(rre_v, [rid, cd])
        rim = plsc.load_gather(rim_v, [rid, cd])
        r = plsc.load_gather(remb_v, [rid, cd])
        res_acc = res_acc + (rre * (erh * ert + eih * eit)
                             + rim * (erh * eit - eih * ert)
                             + eh * et * r)
        vals = (erh, eih, eh, ert, eit, et, rre, rim, r)
        for k in range(9):
          accs[k] = accs[k] + vals[k] * vals[k]
      res_v[pl.ds(j * CHUNK + g * L, L)] = res_acc
    return tuple(accs)

  accs = lax.fori_loop(
      0, NCHUNK, chunk_body,
      tuple(jnp.zeros((L,), jnp.float32) for _ in range(9)))

  for k in range(9):
    reg_v[k, :] = accs[k]
  pltpu.sync_copy(res_v, res_hbm.at[pl.ds(base, BPW)])
  pltpu.sync_copy(reg_v, reg_hbm.at[wid])


@jax.jit
def _sc_call(idh, idt, idr, emb, ere, eim, remb, rre, rim):
  mesh = plsc.VectorSubcoreMesh(core_axis_name="c", subcore_axis_name="s")
  f = pl.kernel(
      _sc_body,
      out_type=(
          jax.ShapeDtypeStruct((BATCH,), jnp.float32),
          jax.ShapeDtypeStruct((NW, 9, HIDDEN), jnp.float32),
      ),
      mesh=mesh,
      scratch_types=[
          pltpu.VMEM((NCHUNK, CHUNK), jnp.int32),
          pltpu.VMEM((NCHUNK, CHUNK), jnp.int32),
          pltpu.VMEM((NCHUNK, CHUNK), jnp.int32),
          pltpu.VMEM((NCHUNK, CHUNK), jnp.int32),
          pltpu.VMEM((NCHUNK, CHUNK), jnp.int32),
      ] + [pltpu.VMEM((CHUNK, 128), jnp.float32) for _ in range(6)] + [
          pltpu.VMEM((REL_TOTAL, HIDDEN), jnp.float32),
          pltpu.VMEM((REL_TOTAL, HIDDEN), jnp.float32),
          pltpu.VMEM((REL_TOTAL, HIDDEN), jnp.float32),
          pltpu.VMEM((BPW,), jnp.float32),
          pltpu.VMEM((9, HIDDEN), jnp.float32),
          pltpu.SemaphoreType.DMA,
      ],
      compiler_params=pltpu.CompilerParams(use_tc_tiling_on_sc=False,
                                           needs_layout_passes=False),
  )
  return f(idh, idt, idr, emb, ere, eim, remb, rre, rim)


def _tc_body(res_ref, y_ref, reg_ref, out_ref):
  x = -(y_ref[...] * res_ref[...])
  sp = jnp.maximum(x, 0.0) + jnp.log(1.0 + jnp.exp(-jnp.abs(x)))
  loss = jnp.sum(sp) * (1.0 / BATCH)
  reg = reg_ref[...]
  scale = 1.0 / (BATCH * HIDDEN)
  m = [jnp.sum(reg[:, k * HIDDEN:(k + 1) * HIDDEN]) * scale for k in range(9)]
  regul = m[0] + m[1] * m[2] + m[3] + m[4] + m[5] + m[6] + m[7] + m[8]
  out_ref[...] = jnp.full((1, 1), loss + LMBDA * regul, jnp.float32)


@jax.jit
def _tc_call(res2, y2, reg2):
  return pl.pallas_call(
      _tc_body,
      out_shape=jax.ShapeDtypeStruct((1, 1), jnp.float32),
  )(res2, y2, reg2)


def kernel(id_h, id_t, id_r, y, ent_embeddings, ent_re, ent_im,
           rel_embeddings, rel_re, rel_im):
  idh = id_h.astype(jnp.int32).reshape(BATCH // CHUNK, CHUNK)
  idt = id_t.astype(jnp.int32).reshape(BATCH // CHUNK, CHUNK)
  idr = id_r.astype(jnp.int32).reshape(BATCH // CHUNK, CHUNK)
  emb2 = ent_embeddings.reshape(ENT_PACKED, 128)
  ere2 = ent_re.reshape(ENT_PACKED, 128)
  eim2 = ent_im.reshape(ENT_PACKED, 128)
  res, regp = _sc_call(idh, idt, idr, emb2, ere2, eim2,
                       rel_embeddings, rel_re, rel_im)
  out = _tc_call(res.reshape(128, 128), y.reshape(128, 128),
                 regp.reshape(NW, 9 * HIDDEN))
  return out[0, 0]


# speedup vs baseline: 4.2618x; 4.2618x over previous
import functools

import jax
import jax.numpy as jnp
from jax import lax
from jax.experimental import pallas as pl
from jax.experimental.pallas import tpu as pltpu
from jax.experimental.pallas import tpu_sc as plsc

ENT_TOTAL = 1000000
REL_TOTAL = 1000
HIDDEN = 16
BATCH = 16384
LMBDA = 0.1

NC = 2
NS = 16
NW = NC * NS
BPW = BATCH // NW
L = 16

NBLK = (ENT_TOTAL + 127) // 128
LAST_BLK = NBLK - 1
CAP = 1280
ORPT = CAP // 8
OUT_ROWS = NW * ORPT
K = 6
NPH = 42
SENT = 2 * BATCH


def _iota():
  return lax.iota(jnp.int32, L)


def _p1_body(idh_hbm, idt_hbm, temb, tre, tim,
             oemb, ore, oim, opos,
             idstage, uns_id, uns_pos, srt_id, srt_pos, cnt_v, start_v,
             sl_emb, sl_re, sl_im, orow_emb, orow_re, orow_im,
             sem_a, sem_b):
  wid = lax.axis_index("s") * NC + lax.axis_index("c")

  def scan_src(src_hbm, tag, off0):
    def chunk(c, off):
      pltpu.sync_copy(src_hbm.at[pl.ds(c * 2048, 2048)], idstage)

      def vec(k, off):
        v = idstage[pl.ds(k * L, L)]
        m = ((v >> 7) & (NW - 1)) == wid
        cnt = plsc.all_reduce_population_count(m)[0]
        plsc.store_compressed(uns_id.at[pl.ds(off, L)], v, mask=m)
        posv = (c * 2048 + tag + k * L) + _iota()
        plsc.store_compressed(uns_pos.at[pl.ds(off, L)], posv, mask=m)
        return off + cnt

      return lax.fori_loop(0, 2048 // L, vec, off)

    return lax.fori_loop(0, BATCH // 2048, chunk, off0)

  n = scan_src(idh_hbm, 0, jnp.int32(0))
  n = scan_src(idt_hbm, BATCH, n)

  zero16 = jnp.zeros((L,), jnp.int32)
  for t in range(272 // L):
    cnt_v[pl.ds(t * L, L)] = zero16
    srt_pos[pl.ds(t * L, L)] = zero16 + SENT
  for t in range(272 // L, (CAP + L) // L):
    srt_pos[pl.ds(t * L, L)] = zero16 + SENT

  ones16 = jnp.ones((L,), jnp.int32)

  def count_vec(kv, _):
    mval = (kv * L + _iota()) < n
    j = uns_id[pl.ds(kv * L, L)] >> 12
    j = jnp.where(mval, j, 270)
    plsc.addupdate_scatter(cnt_v, [j], ones16, mask=mval)
    return 0

  lax.fori_loop(0, (CAP + L) // L, count_vec, 0)

  run = jnp.int32(0)
  for t in range(272 // L):
    v = cnt_v[pl.ds(t * L, L)]
    cs = plsc.cumsum(v)
    start_v[pl.ds(t * L, L)] = run + cs - v
    run = run + cs[L - 1]
  for t in range(272 // L):
    cnt_v[pl.ds(t * L, L)] = start_v[pl.ds(t * L, L)]

  def place_vec(kv, _):
    lanes = kv * L + _iota()
    mval = lanes < n
    idv = uns_id[pl.ds(kv * L, L)]
    posv = uns_pos[pl.ds(kv * L, L)]
    j = jnp.where(mval, idv >> 12, 270)
    cur = plsc.load_gather(cnt_v, [j])
    mi = mval.astype(jnp.int32)
    ordv = jnp.zeros((L,), jnp.int32)
    for l in range(L):
      same = (j == j[l]) & (_iota() > l)
      ordv = ordv + same.astype(jnp.int32) * mi[l]
    slot = cur + ordv
    plsc.store_scatter(srt_id, [slot], idv, mask=mval)
    plsc.store_scatter(srt_pos, [slot], posv, mask=mval)
    plsc.addupdate_scatter(cnt_v, [j], ones16, mask=mval)
    return 0

  lax.fori_loop(0, (CAP + L) // L, place_vec, 0)

  tables = ((temb, sl_emb, orow_emb, oemb),
            (tre, sl_re, orow_re, ore),
            (tim, sl_im, orow_im, oim))

  def dma_phase(ph, g, fire):
    sem = sem_a if g == 0 else sem_b
    for b in range(K):
      j = ph * K + b
      blk = wid + NW * j
      col = pl.multiple_of(jnp.where(blk <= LAST_BLK, blk, 0) * 128, 128)
      for tbl, sl, _o, _oh in tables:
        cp = pltpu.make_async_copy(tbl.at[:, pl.ds(col, 128)],
                                   sl.at[g].at[b], sem)
        if fire:
          cp.start()
        else:
          cp.wait()

  def process_phase(ph, g):
    for b in range(K):
      j = ph * K + b
      lo = start_v[pl.ds(j, L)][0]
      hi = start_v[pl.ds(j + 1, L)][0]

      def entry(e, _):
        idv = srt_id[pl.ds(e, L)]
        loff = idv[0] & 127
        coli = jnp.full((L,), loff, jnp.int32)
        orow = (e >> 3) & 7
        osl = pl.ds((e & 7) * HIDDEN, HIDDEN)
        for _tbl, sl, orow_v, _oh in tables:
          v = plsc.load_gather(sl.at[g].at[b], [_iota(), coli])
          orow_v[orow, osl] = v

        @pl.when((e & 63) == 63)
        def _():
          rb = pl.multiple_of(wid * ORPT + ((e >> 6) << 3), 8)
          for _tbl, _sl, orow_v, out_hbm in tables:
            pltpu.sync_copy(orow_v, out_hbm.at[pl.ds(rb, 8)])

        return 0

      lax.fori_loop(lo, hi, entry, 0)

  dma_phase(0, 0, True)

  def pair(p, _):
    dma_phase(2 * p + 1, 1, True)
    dma_phase(2 * p, 0, False)
    process_phase(2 * p, 0)

    @pl.when(p < NPH // 2 - 1)
    def _():
      dma_phase(2 * p + 2, 0, True)

    dma_phase(2 * p + 1, 1, False)
    process_phase(2 * p + 1, 1)
    return 0

  lax.fori_loop(0, NPH // 2, pair, 0)

  @pl.when(n > 0)
  def _():
    rb = pl.multiple_of(wid * ORPT + (((n - 1) >> 6) << 3), 8)
    for _tbl, _sl, orow_v, out_hbm in tables:
      pltpu.sync_copy(orow_v, out_hbm.at[pl.ds(rb, 8)])

  pltpu.sync_copy(srt_pos.at[pl.ds(0, CAP)], opos.at[pl.ds(wid * CAP, CAP)])


def _p2_body(opos_hbm, idr_hbm, oemb, ore, oim, remb_hbm, rre_hbm, rim_hbm,
             res_hbm, reg_hbm,
             posstage, inv_v, idr_v, sh_v, st_v, ixh, ixt,
             g_eh, g_ert2, g_et, g_erh2, g_eih2, g_eit2,
             remb_v, rre_v, rim_v, res_v, reg_v, sem):
  wid = lax.axis_index("s") * NC + lax.axis_index("c")
  base = wid * BPW

  pltpu.sync_copy(idr_hbm.at[pl.ds(base, BPW)], idr_v)
  pltpu.sync_copy(remb_hbm, remb_v)
  pltpu.sync_copy(rre_hbm, rre_v)
  pltpu.sync_copy(rim_hbm, rim_v)

  def pchunk(cc, _):
    pltpu.sync_copy(opos_hbm.at[pl.ds(cc * 2048, 2048)], posstage)

    def vec(kv, _):
      posv = posstage[pl.ds(kv * L, L)]
      slotv = cc * 2048 + kv * L + _iota()
      in_h = (posv >= base) & (posv < base + BPW)
      in_t = (posv >= BATCH + base) & (posv < BATCH + base + BPW)
      m = in_h | in_t
      idx = jnp.where(in_h, posv - base, posv - (BATCH + base) + BPW)
      idx = jnp.where(m, idx, 2 * BPW)
      plsc.store_scatter(inv_v, [idx], slotv, mask=m)
      return 0

    lax.fori_loop(0, 2048 // L, vec, 0)
    return 0

  lax.fori_loop(0, (NW * CAP) // 2048, pchunk, 0)

  for k in range(BPW // L):
    sh = inv_v[pl.ds(k * L, L)]
    st = inv_v[pl.ds(BPW + k * L, L)]
    sh_v[pl.ds(k * L, L)] = sh
    st_v[pl.ds(k * L, L)] = st
    ixh[k // 4, pl.ds((k % 4) * L, L)] = sh >> 3
    ixt[k // 4, pl.ds((k % 4) * L, L)] = st >> 3

  accs = [jnp.zeros((L,), jnp.float32) for _ in range(9)]

  for ci in range(BPW // 64):
    cps = [
        pltpu.async_copy(oemb.at[ixh.at[ci]], g_eh, sem),
        pltpu.async_copy(ore.at[ixh.at[ci]], g_erh2, sem),
        pltpu.async_copy(oim.at[ixh.at[ci]], g_eih2, sem),
        pltpu.async_copy(oemb.at[ixt.at[ci]], g_et, sem),
        pltpu.async_copy(ore.at[ixt.at[ci]], g_ert2, sem),
        pltpu.async_copy(oim.at[ixt.at[ci]], g_eit2, sem),
    ]
    for cp in cps:
      cp.wait()

    for gidx in range(64 // L):
      r0 = ci * 64 + gidx * L
      s = pl.ds(r0, L)
      offh = (sh_v[s] & 7) * HIDDEN
      offt = (st_v[s] & 7) * HIDDEN
      rid = idr_v[s]
      rowi = jnp.arange(L, dtype=jnp.int32) + (gidx * L)
      res_acc = jnp.zeros((L,), jnp.float32)
      for d in range(HIDDEN):
        ch = offh + d
        ct = offt + d
        cd = jnp.full((L,), d, jnp.int32)
        erh = plsc.load_gather(g_erh2, [rowi, ch])
        eih = plsc.load_gather(g_eih2, [rowi, ch])
        eh = plsc.load_gather(g_eh, [rowi, ch])
        ert = plsc.load_gather(g_ert2, [rowi, ct])
        eit = plsc.load_gather(g_eit2, [rowi, ct])
        et = plsc.load_gather(g_et, [rowi, ct])
        rre = plsc.load_gather(rre_v, [rid, cd])
        rim = plsc.load_gather(rim_v, [rid, cd])
        r = plsc.load_gather(remb_v, [rid, cd])
        res_acc = res_acc + (rre * (erh * ert + eih * eit)
                             + rim * (erh * eit - eih * ert)
                             + eh * et * r)
        vals = (erh, eih, eh, ert, eit, et, rre, rim, r)
        for q in range(9):
          accs[q] = accs[q] + vals[q] * vals[q]
      res_v[pl.ds(r0, L)] = res_acc

  for q in range(9):
    reg_v[q, :] = accs[q]
  pltpu.sync_copy(res_v, res_hbm.at[pl.ds(base, BPW)])
  pltpu.sync_copy(reg_v, reg_hbm.at[wid])


@jax.jit
def _sc_calls(idh, idt, idr, emb, ere, eim, remb, rre, rim):
  mesh = plsc.VectorSubcoreMesh(core_axis_name="c", subcore_axis_name="s")
  p1 = pl.kernel(
      _p1_body,
      out_type=(
          jax.ShapeDtypeStruct((OUT_ROWS, 128), jnp.float32),
          jax.ShapeDtypeStruct((OUT_ROWS, 128), jnp.float32),
          jax.ShapeDtypeStruct((OUT_ROWS, 128), jnp.float32),
          jax.ShapeDtypeStruct((NW * CAP,), jnp.int32),
      ),
      mesh=mesh,
      scratch_types=[
          pltpu.VMEM((2048,), jnp.int32),
          pltpu.VMEM((CAP + L,), jnp.int32),
          pltpu.VMEM((CAP + L,), jnp.int32),
          pltpu.VMEM((CAP + L,), jnp.int32),
          pltpu.VMEM((CAP + L,), jnp.int32),
          pltpu.VMEM((272,), jnp.int32),
          pltpu.VMEM((272,), jnp.int32),
          pltpu.VMEM((2, K, L, 128), jnp.float32),
          pltpu.VMEM((2, K, L, 128), jnp.float32),
          pltpu.VMEM((2, K, L, 128), jnp.float32),
          pltpu.VMEM((8, 128), jnp.float32),
          pltpu.VMEM((8, 128), jnp.float32),
          pltpu.VMEM((8, 128), jnp.float32),
          pltpu.SemaphoreType.DMA,
          pltpu.SemaphoreType.DMA,
      ],
      compiler_params=pltpu.CompilerParams(use_tc_tiling_on_sc=True,
                                           needs_layout_passes=False),
  )
  vemb, vre, vim, pos = p1(idh, idt, emb.T, ere.T, eim.T)

  p2 = pl.kernel(
      _p2_body,
      out_type=(
          jax.ShapeDtypeStruct((BATCH,), jnp.float32),
          jax.ShapeDtypeStruct((NW, 9, HIDDEN), jnp.float32),
      ),
      mesh=mesh,
      scratch_types=[
          pltpu.VMEM((2048,), jnp.int32),
          pltpu.VMEM((2 * BPW + L,), jnp.int32),
          pltpu.VMEM((BPW,), jnp.int32),
          pltpu.VMEM((BPW,), jnp.int32),
          pltpu.VMEM((BPW,), jnp.int32),
          pltpu.VMEM((BPW // 64, 64), jnp.int32),
          pltpu.VMEM((BPW // 64, 64), jnp.int32),
      ] + [pltpu.VMEM((64, 128), jnp.float32) for _ in range(6)] + [
          pltpu.VMEM((REL_TOTAL, HIDDEN), jnp.float32),
          pltpu.VMEM((REL_TOTAL, HIDDEN), jnp.float32),
          pltpu.VMEM((REL_TOTAL, HIDDEN), jnp.float32),
          pltpu.VMEM((BPW,), jnp.float32),
          pltpu.VMEM((9, HIDDEN), jnp.float32),
          pltpu.SemaphoreType.DMA,
      ],
      compiler_params=pltpu.CompilerParams(use_tc_tiling_on_sc=False,
                                           needs_layout_passes=False),
  )
  return p2(pos, idr, vemb, vre, vim, remb, rre, rim)


def _tc_body(res_ref, y_ref, reg_ref, out_ref):
  x = -(y_ref[...] * res_ref[...])
  sp = jnp.maximum(x, 0.0) + jnp.log(1.0 + jnp.exp(-jnp.abs(x)))
  loss = jnp.sum(sp) * (1.0 / BATCH)
  reg = reg_ref[...]
  scale = 1.0 / (BATCH * HIDDEN)
  m = [jnp.sum(reg[:, q * HIDDEN:(q + 1) * HIDDEN]) * scale for q in range(9)]
  regul = m[0] + m[1] * m[2] + m[3] + m[4] + m[5] + m[6] + m[7] + m[8]
  out_ref[...] = jnp.full((1, 1), loss + LMBDA * regul, jnp.float32)


@jax.jit
def _tc_call(res2, y2, reg2):
  return pl.pallas_call(
      _tc_body,
      out_shape=jax.ShapeDtypeStruct((1, 1), jnp.float32),
  )(res2, y2, reg2)


def kernel(id_h, id_t, id_r, y, ent_embeddings, ent_re, ent_im,
           rel_embeddings, rel_re, rel_im):
  idh = id_h.astype(jnp.int32)
  idt = id_t.astype(jnp.int32)
  idr = id_r.astype(jnp.int32)
  res, regp = _sc_calls(idh, idt, idr, ent_embeddings, ent_re, ent_im,
                        rel_embeddings, rel_re, rel_im)
  out = _tc_call(res.reshape(128, 128), y.reshape(128, 128),
                 regp.reshape(NW, 9 * HIDDEN))
  return out[0, 0]


# speedup vs baseline: 4.9223x; 1.1550x over previous
import functools

import jax
import jax.numpy as jnp
from jax import lax
from jax.experimental import pallas as pl
from jax.experimental.pallas import tpu as pltpu
from jax.experimental.pallas import tpu_sc as plsc

ENT_TOTAL = 1000000
REL_TOTAL = 1000
HIDDEN = 16
BATCH = 16384
LMBDA = 0.1

NC = 2
NS = 16
NW = NC * NS
BPW = BATCH // NW
L = 16

NBLK = (ENT_TOTAL + 127) // 128
LAST_BLK = NBLK - 1
CAP = 1280
ORPT = CAP // 8
OUT_ROWS = NW * ORPT
K = 6
NPH = 42
SENT = 2 * BATCH


def _iota():
  return lax.iota(jnp.int32, L)


def _p1_body(idh_hbm, idt_hbm, temb, tre, tim,
             oemb, ore, oim, opos,
             idstage, uns_id, uns_pos, srt_id, srt_pos, cnt_v, start_v,
             sl_emb, sl_re, sl_im, orow_emb, orow_re, orow_im,
             sem_a, sem_b, sem_f):
  wid = lax.axis_index("s") * NC + lax.axis_index("c")

  def scan_src(src_hbm, tag, off0):
    def chunk(c, off):
      pltpu.sync_copy(src_hbm.at[pl.ds(c * 2048, 2048)], idstage)

      def vec(k, off):
        v = idstage[pl.ds(k * L, L)]
        m = ((v >> 7) & (NW - 1)) == wid
        cnt = plsc.all_reduce_population_count(m)[0]
        plsc.store_compressed(uns_id.at[pl.ds(off, L)], v, mask=m)
        posv = (c * 2048 + tag + k * L) + _iota()
        plsc.store_compressed(uns_pos.at[pl.ds(off, L)], posv, mask=m)
        return off + cnt

      return lax.fori_loop(0, 2048 // L, vec, off)

    return lax.fori_loop(0, BATCH // 2048, chunk, off0)

  n = scan_src(idh_hbm, 0, jnp.int32(0))
  n = scan_src(idt_hbm, BATCH, n)

  zero16 = jnp.zeros((L,), jnp.int32)
  for t in range(272 // L):
    cnt_v[pl.ds(t * L, L)] = zero16
    srt_pos[pl.ds(t * L, L)] = zero16 + SENT
  for t in range(272 // L, (CAP + L) // L):
    srt_pos[pl.ds(t * L, L)] = zero16 + SENT

  ones16 = jnp.ones((L,), jnp.int32)

  def count_vec(kv, _):
    mval = (kv * L + _iota()) < n
    j = uns_id[pl.ds(kv * L, L)] >> 12
    j = jnp.where(mval, j, 270)
    plsc.addupdate_scatter(cnt_v, [j], ones16, mask=mval)
    return 0

  lax.fori_loop(0, (CAP + L) // L, count_vec, 0)

  run = jnp.int32(0)
  for t in range(272 // L):
    v = cnt_v[pl.ds(t * L, L)]
    cs = plsc.cumsum(v)
    start_v[pl.ds(t * L, L)] = run + cs - v
    run = run + cs[L - 1]
  for t in range(272 // L):
    cnt_v[pl.ds(t * L, L)] = start_v[pl.ds(t * L, L)]

  def place_vec(kv, _):
    lanes = kv * L + _iota()
    mval = lanes < n
    idv = uns_id[pl.ds(kv * L, L)]
    posv = uns_pos[pl.ds(kv * L, L)]
    j = jnp.where(mval, idv >> 12, 270)
    cur = plsc.load_gather(cnt_v, [j])
    mi = mval.astype(jnp.int32)
    ordv = jnp.zeros((L,), jnp.int32)
    for l in range(L):
      same = (j == j[l]) & (_iota() > l)
      ordv = ordv + same.astype(jnp.int32) * mi[l]
    slot = cur + ordv
    plsc.store_scatter(srt_id, [slot], idv, mask=mval)
    plsc.store_scatter(srt_pos, [slot], posv, mask=mval)
    plsc.addupdate_scatter(cnt_v, [j], ones16, mask=mval)
    return 0

  lax.fori_loop(0, (CAP + L) // L, place_vec, 0)

  tables = ((temb, sl_emb, orow_emb, oemb),
            (tre, sl_re, orow_re, ore),
            (tim, sl_im, orow_im, oim))

  def dma_phase(ph, g, fire):
    sem = sem_a if g == 0 else sem_b
    for b in range(K):
      j = ph * K + b
      blk = wid + NW * j
      col = pl.multiple_of(jnp.where(blk <= LAST_BLK, blk, 0) * 128, 128)
      for tbl, sl, _o, _oh in tables:
        cp = pltpu.make_async_copy(tbl.at[:, pl.ds(col, 128)],
                                   sl.at[g].at[b], sem)
        if fire:
          cp.start()
        else:
          cp.wait()

  def process_phase(ph, g):
    for b in range(K):
      j = ph * K + b
      lo = start_v[pl.ds(j, L)][0]
      hi = start_v[pl.ds(j + 1, L)][0]

      def entry(e, _):
        idv = srt_id[pl.ds(e, L)]
        loff = idv[0] & 127
        coli = jnp.full((L,), loff, jnp.int32)
        orow = (e >> 3) & 7
        osl = pl.ds((e & 7) * HIDDEN, HIDDEN)
        for _tbl, sl, orow_v, _oh in tables:
          v = plsc.load_gather(sl.at[g].at[b], [_iota(), coli])
          orow_v[orow, osl] = v

        @pl.when((e & 63) == 63)
        def _():
          rb = pl.multiple_of(wid * ORPT + ((e >> 6) << 3), 8)
          fcps = [pltpu.async_copy(orow_v, out_hbm.at[pl.ds(rb, 8)], sem_f)
                  for _tbl, _sl, orow_v, out_hbm in tables]
          for fcp in fcps:
            fcp.wait()

        return 0

      lax.fori_loop(lo, hi, entry, 0)

  dma_phase(0, 0, True)

  def pair(p, _):
    dma_phase(2 * p + 1, 1, True)
    dma_phase(2 * p, 0, False)
    process_phase(2 * p, 0)

    @pl.when(p < NPH // 2 - 1)
    def _():
      dma_phase(2 * p + 2, 0, True)

    dma_phase(2 * p + 1, 1, False)
    process_phase(2 * p + 1, 1)
    return 0

  lax.fori_loop(0, NPH // 2, pair, 0)

  @pl.when(n > 0)
  def _():
    rb = pl.multiple_of(wid * ORPT + (((n - 1) >> 6) << 3), 8)
    fcps = [pltpu.async_copy(orow_v, out_hbm.at[pl.ds(rb, 8)], sem_f)
            for _tbl, _sl, orow_v, out_hbm in tables]
    for fcp in fcps:
      fcp.wait()

  pltpu.sync_copy(srt_pos.at[pl.ds(0, CAP)], opos.at[pl.ds(wid * CAP, CAP)])


def _p2_body(opos_hbm, idr_hbm, oemb, ore, oim, remb_hbm, rre_hbm, rim_hbm,
             res_hbm, reg_hbm,
             posall, inv_v, idr_v, ixh, ixt,
             gbufs, remb_v, rre_v, rim_v, res_v, reg_v, sem, semr):
  wid = lax.axis_index("s") * NC + lax.axis_index("c")
  base = wid * BPW

  pltpu.sync_copy(opos_hbm, posall)
  pltpu.sync_copy(idr_hbm.at[pl.ds(base, BPW)], idr_v)
  rel_cps = [pltpu.async_copy(remb_hbm, remb_v, semr),
             pltpu.async_copy(rre_hbm, rre_v, semr),
             pltpu.async_copy(rim_hbm, rim_v, semr)]

  def vec(kv, _):
    posv = posall[pl.ds(kv * L, L)]
    slotv = kv * L + _iota()
    in_h = (posv >= base) & (posv < base + BPW)
    in_t = (posv >= BATCH + base) & (posv < BATCH + base + BPW)
    m = in_h | in_t
    idx = jnp.where(in_h, posv - base, posv - (BATCH + base) + BPW)
    idx = jnp.where(m, idx, 2 * BPW)
    plsc.store_scatter(inv_v, [idx], slotv, mask=m)
    return 0

  lax.fori_loop(0, (NW * CAP) // L, vec, 0)

  for k in range(BPW // L):
    sh = inv_v[pl.ds(k * L, L)]
    st = inv_v[pl.ds(BPW + k * L, L)]
    ixh[k // 4, pl.ds((k % 4) * L, L)] = sh
    ixt[k // 4, pl.ds((k % 4) * L, L)] = st

  for cp in rel_cps:
    cp.wait()

  NCH = BPW // 64

  def fire(ci):
    g = gbufs[ci % 2]
    return [
        pltpu.async_copy(oemb.at[ixh.at[ci]], g[0], sem),
        pltpu.async_copy(ore.at[ixh.at[ci]], g[1], sem),
        pltpu.async_copy(oim.at[ixh.at[ci]], g[2], sem),
        pltpu.async_copy(oemb.at[ixt.at[ci]], g[3], sem),
        pltpu.async_copy(ore.at[ixt.at[ci]], g[4], sem),
        pltpu.async_copy(oim.at[ixt.at[ci]], g[5], sem),
    ]

  accs = [jnp.zeros((L,), jnp.float32) for _ in range(9)]
  cps = fire(0)
  for ci in range(NCH):
    for cp in cps:
      cp.wait()
    if ci + 1 < NCH:
      cps = fire(ci + 1)
    g_eh, g_erh2, g_eih2, g_et, g_ert2, g_eit2 = gbufs[ci % 2]

    for gidx in range(64 // L):
      r0 = ci * 64 + gidx * L
      sl = pl.ds(r0, L)
      rid = idr_v[sl]
      rowi = jnp.arange(L, dtype=jnp.int32) + (gidx * L)
      res_acc = jnp.zeros((L,), jnp.float32)
      for d in range(HIDDEN):
        cd = jnp.full((L,), d, jnp.int32)
        erh = plsc.load_gather(g_erh2, [rowi, cd])
        eih = plsc.load_gather(g_eih2, [rowi, cd])
        eh = plsc.load_gather(g_eh, [rowi, cd])
        ert = plsc.load_gather(g_ert2, [rowi, cd])
        eit = plsc.load_gather(g_eit2, [rowi, cd])
        et = plsc.load_gather(g_et, [rowi, cd])
        rre = plsc.load_gather(rre_v, [rid, cd])
        rim = plsc.load_gather(rim_v, [rid, cd])
        r = plsc.load_gather(remb_v, [rid, cd])
        res_acc = res_acc + (rre * (erh * ert + eih * eit)
                             + rim * (erh * eit - eih * ert)
                             + eh * et * r)
        vals = (erh, eih, eh, ert, eit, et, rre, rim, r)
        for q in range(9):
          accs[q] = accs[q] + vals[q] * vals[q]
      res_v[pl.ds(r0, L)] = res_acc

  for q in range(9):
    reg_v[q, :] = accs[q]
  pltpu.sync_copy(res_v, res_hbm.at[pl.ds(base, BPW)])
  pltpu.sync_copy(reg_v, reg_hbm.at[wid])


@jax.jit
def _sc_calls(idh, idt, idr, emb, ere, eim, remb, rre, rim):
  mesh = plsc.VectorSubcoreMesh(core_axis_name="c", subcore_axis_name="s")
  p1 = pl.kernel(
      _p1_body,
      out_type=(
          jax.ShapeDtypeStruct((OUT_ROWS, 128), jnp.float32),
          jax.ShapeDtypeStruct((OUT_ROWS, 128), jnp.float32),
          jax.ShapeDtypeStruct((OUT_ROWS, 128), jnp.float32),
          jax.ShapeDtypeStruct((NW * CAP,), jnp.int32),
      ),
      mesh=mesh,
      scratch_types=[
          pltpu.VMEM((2048,), jnp.int32),
          pltpu.VMEM((CAP + L,), jnp.int32),
          pltpu.VMEM((CAP + L,), jnp.int32),
          pltpu.VMEM((CAP + L,), jnp.int32),
          pltpu.VMEM((CAP + L,), jnp.int32),
          pltpu.VMEM((272,), jnp.int32),
          pltpu.VMEM((272,), jnp.int32),
          pltpu.VMEM((2, K, L, 128), jnp.float32),
          pltpu.VMEM((2, K, L, 128), jnp.float32),
          pltpu.VMEM((2, K, L, 128), jnp.float32),
          pltpu.VMEM((8, 128), jnp.float32),
          pltpu.VMEM((8, 128), jnp.float32),
          pltpu.VMEM((8, 128), jnp.float32),
          pltpu.SemaphoreType.DMA,
          pltpu.SemaphoreType.DMA,
          pltpu.SemaphoreType.DMA,
      ],
      compiler_params=pltpu.CompilerParams(use_tc_tiling_on_sc=True,
                                           needs_layout_passes=False),
  )
  vemb, vre, vim, pos = p1(idh, idt, emb.T, ere.T, eim.T)

  p2 = pl.kernel(
      _p2_body,
      out_type=(
          jax.ShapeDtypeStruct((BATCH,), jnp.float32),
          jax.ShapeDtypeStruct((NW, 9, HIDDEN), jnp.float32),
      ),
      mesh=mesh,
      scratch_types=[
          pltpu.VMEM((NW * CAP,), jnp.int32),
          pltpu.VMEM((2 * BPW + L,), jnp.int32),
          pltpu.VMEM((BPW,), jnp.int32),
          pltpu.VMEM((BPW // 64, 64), jnp.int32),
          pltpu.VMEM((BPW // 64, 64), jnp.int32),
          [[pltpu.VMEM((64, HIDDEN), jnp.float32) for _ in range(6)]
           for _ in range(2)],
          pltpu.VMEM((REL_TOTAL, HIDDEN), jnp.float32),
          pltpu.VMEM((REL_TOTAL, HIDDEN), jnp.float32),
          pltpu.VMEM((REL_TOTAL, HIDDEN), jnp.float32),
          pltpu.VMEM((BPW,), jnp.float32),
          pltpu.VMEM((9, HIDDEN), jnp.float32),
          pltpu.SemaphoreType.DMA,
          pltpu.SemaphoreType.DMA,
      ],
      compiler_params=pltpu.CompilerParams(use_tc_tiling_on_sc=False,
                                           needs_layout_passes=False),
  )
  return p2(pos, idr, vemb.reshape(NW * CAP, HIDDEN),
            vre.reshape(NW * CAP, HIDDEN),
            vim.reshape(NW * CAP, HIDDEN), remb, rre, rim)


def _tc_body(res_ref, y_ref, reg_ref, out_ref):
  x = -(y_ref[...] * res_ref[...])
  sp = jnp.maximum(x, 0.0) + jnp.log(1.0 + jnp.exp(-jnp.abs(x)))
  loss = jnp.sum(sp) * (1.0 / BATCH)
  reg = reg_ref[...]
  scale = 1.0 / (BATCH * HIDDEN)
  m = [jnp.sum(reg[:, q * HIDDEN:(q + 1) * HIDDEN]) * scale for q in range(9)]
  regul = m[0] + m[1] * m[2] + m[3] + m[4] + m[5] + m[6] + m[7] + m[8]
  out_ref[...] = jnp.full((1, 1), loss + LMBDA * regul, jnp.float32)


@jax.jit
def _tc_call(res2, y2, reg2):
  return pl.pallas_call(
      _tc_body,
      out_shape=jax.ShapeDtypeStruct((1, 1), jnp.float32),
  )(res2, y2, reg2)


def kernel(id_h, id_t, id_r, y, ent_embeddings, ent_re, ent_im,
           rel_embeddings, rel_re, rel_im):
  idh = id_h.astype(jnp.int32)
  idt = id_t.astype(jnp.int32)
  idr = id_r.astype(jnp.int32)
  res, regp = _sc_calls(idh, idt, idr, ent_embeddings, ent_re, ent_im,
                        rel_embeddings, rel_re, rel_im)
  out = _tc_call(res.reshape(128, 128), y.reshape(128, 128),
                 regp.reshape(NW, 9 * HIDDEN))
  return out[0, 0]


# speedup vs baseline: 5.0513x; 1.0262x over previous
import functools

import jax
import jax.numpy as jnp
from jax import lax
from jax.experimental import pallas as pl
from jax.experimental.pallas import tpu as pltpu
from jax.experimental.pallas import tpu_sc as plsc

ENT_TOTAL = 1000000
REL_TOTAL = 1000
HIDDEN = 16
BATCH = 16384
LMBDA = 0.1

NC = 2
NS = 16
NW = NC * NS
BPW = BATCH // NW
L = 16

NBLK = (ENT_TOTAL + 127) // 128
LAST_BLK = NBLK - 1
CAP = 1280
ORPT = CAP // 8
OUT_ROWS = NW * ORPT
K = 8
NPH = 32
SENT = 2 * BATCH


def _iota():
  return lax.iota(jnp.int32, L)


def _p1_body(idh_hbm, idt_hbm, temb, tre, tim,
             oemb, ore, oim, opos,
             idstage, uns_id, uns_pos, srt_id, srt_pos, cnt_v, start_v,
             sl_emb, sl_re, sl_im, orow_emb, orow_re, orow_im,
             sem_a, sem_b, sem_f):
  wid = lax.axis_index("s") * NC + lax.axis_index("c")

  def scan_src(src_hbm, tag, off0):
    def chunk(c, off):
      pltpu.sync_copy(src_hbm.at[pl.ds(c * 2048, 2048)], idstage)

      def vec4(k4, off):
        vs, ms, cnts = [], [], []
        for u in range(4):
          v = idstage[pl.ds((k4 * 4 + u) * L, L)]
          m = ((v >> 7) & (NW - 1)) == wid
          vs.append(v)
          ms.append(m)
          cnts.append(plsc.all_reduce_population_count(m)[0])
        for u in range(4):
          plsc.store_compressed(uns_id.at[pl.ds(off, L)], vs[u], mask=ms[u])
          posv = (c * 2048 + tag + (k4 * 4 + u) * L) + _iota()
          plsc.store_compressed(uns_pos.at[pl.ds(off, L)], posv, mask=ms[u])
          off = off + cnts[u]
        return off

      return lax.fori_loop(0, 2048 // L // 4, vec4, off)

    return lax.fori_loop(0, BATCH // 2048, chunk, off0)

  n = scan_src(idh_hbm, 0, jnp.int32(0))
  n = scan_src(idt_hbm, BATCH, n)

  zero16 = jnp.zeros((L,), jnp.int32)
  for t in range(272 // L):
    cnt_v[pl.ds(t * L, L)] = zero16
    srt_pos[pl.ds(t * L, L)] = zero16 + SENT
  for t in range(272 // L, (CAP + L) // L):
    srt_pos[pl.ds(t * L, L)] = zero16 + SENT

  ones16 = jnp.ones((L,), jnp.int32)

  def count_vec(kv, _):
    mval = (kv * L + _iota()) < n
    j = uns_id[pl.ds(kv * L, L)] >> 12
    j = jnp.where(mval, j, 270)
    plsc.addupdate_scatter(cnt_v, [j], ones16, mask=mval)
    return 0

  lax.fori_loop(0, (CAP + L) // L, count_vec, 0)

  run = jnp.int32(0)
  for t in range(272 // L):
    v = cnt_v[pl.ds(t * L, L)]
    cs = plsc.cumsum(v)
    start_v[pl.ds(t * L, L)] = run + cs - v
    run = run + cs[L - 1]
  for t in range(272 // L):
    cnt_v[pl.ds(t * L, L)] = start_v[pl.ds(t * L, L)]

  def place_vec(kv, _):
    lanes = kv * L + _iota()
    mval = lanes < n
    idv = uns_id[pl.ds(kv * L, L)]
    posv = uns_pos[pl.ds(kv * L, L)]
    j = jnp.where(mval, idv >> 12, 270)
    cur = plsc.load_gather(cnt_v, [j])
    mi = mval.astype(jnp.int32)
    ordv = jnp.zeros((L,), jnp.int32)
    for l in range(L):
      same = (j == j[l]) & (_iota() > l)
      ordv = ordv + same.astype(jnp.int32) * mi[l]
    slot = cur + ordv
    plsc.store_scatter(srt_id, [slot], idv, mask=mval)
    plsc.store_scatter(srt_pos, [slot], posv, mask=mval)
    plsc.addupdate_scatter(cnt_v, [j], ones16, mask=mval)
    return 0

  lax.fori_loop(0, (CAP + L) // L, place_vec, 0)

  tables = ((temb, sl_emb, orow_emb, oemb),
            (tre, sl_re, orow_re, ore),
            (tim, sl_im, orow_im, oim))

  def dma_phase(ph, g, fire):
    sem = sem_a if g == 0 else sem_b
    for b in range(K):
      j = ph * K + b
      blk = wid + NW * j
      col = pl.multiple_of(jnp.where(blk <= LAST_BLK, blk, 0) * 128, 128)
      for tbl, sl, _o, _oh in tables:
        cp = pltpu.make_async_copy(tbl.at[:, pl.ds(col, 128)],
                                   sl.at[g].at[b], sem)
        if fire:
          cp.start()
        else:
          cp.wait()

  def process_phase(ph, g):
    for b in range(K):
      j = ph * K + b
      lo = start_v[pl.ds(j, L)][0]
      hi = start_v[pl.ds(j + 1, L)][0]

      def entry(e, _):
        idv = srt_id[pl.ds(e, L)]
        loff = idv[0] & 127
        coli = jnp.full((L,), loff, jnp.int32)
        orow = (e >> 3) & 7
        osl = pl.ds((e & 7) * HIDDEN, HIDDEN)
        for _tbl, sl, orow_v, _oh in tables:
          v = plsc.load_gather(sl.at[g].at[b], [_iota(), coli])
          orow_v[orow, osl] = v

        @pl.when((e & 63) == 63)
        def _():
          rb = pl.multiple_of(wid * ORPT + ((e >> 6) << 3), 8)
          fcps = [pltpu.async_copy(orow_v, out_hbm.at[pl.ds(rb, 8)], sem_f)
                  for _tbl, _sl, orow_v, out_hbm in tables]
          for fcp in fcps:
            fcp.wait()

        return 0

      lax.fori_loop(lo, hi, entry, 0)

  dma_phase(0, 0, True)

  def pair(p, _):
    dma_phase(2 * p + 1, 1, True)
    dma_phase(2 * p, 0, False)
    process_phase(2 * p, 0)

    @pl.when(p < NPH // 2 - 1)
    def _():
      dma_phase(2 * p + 2, 0, True)

    dma_phase(2 * p + 1, 1, False)
    process_phase(2 * p + 1, 1)
    return 0

  lax.fori_loop(0, NPH // 2, pair, 0)

  @pl.when(n > 0)
  def _():
    rb = pl.multiple_of(wid * ORPT + (((n - 1) >> 6) << 3), 8)
    fcps = [pltpu.async_copy(orow_v, out_hbm.at[pl.ds(rb, 8)], sem_f)
            for _tbl, _sl, orow_v, out_hbm in tables]
    for fcp in fcps:
      fcp.wait()

  pltpu.sync_copy(srt_pos.at[pl.ds(0, CAP)], opos.at[pl.ds(wid * CAP, CAP)])


def _p2_body(opos_hbm, idr_hbm, oemb, ore, oim, remb_hbm, rre_hbm, rim_hbm,
             res_hbm, reg_hbm,
             posall, inv_v, idr_v, ixh, ixt,
             gbufs, remb_v, rre_v, rim_v, res_v, reg_v, sem, semr):
  wid = lax.axis_index("s") * NC + lax.axis_index("c")
  base = wid * BPW

  pltpu.sync_copy(opos_hbm, posall)
  pltpu.sync_copy(idr_hbm.at[pl.ds(base, BPW)], idr_v)
  rel_cps = [pltpu.async_copy(remb_hbm, remb_v, semr),
             pltpu.async_copy(rre_hbm, rre_v, semr),
             pltpu.async_copy(rim_hbm, rim_v, semr)]

  def vec(kv, _):
    posv = posall[pl.ds(kv * L, L)]
    slotv = kv * L + _iota()
    in_h = (posv >= base) & (posv < base + BPW)
    in_t = (posv >= BATCH + base) & (posv < BATCH + base + BPW)
    m = in_h | in_t
    idx = jnp.where(in_h, posv - base, posv - (BATCH + base) + BPW)
    idx = jnp.where(m, idx, 2 * BPW)
    plsc.store_scatter(inv_v, [idx], slotv, mask=m)
    return 0

  lax.fori_loop(0, (NW * CAP) // L, vec, 0)

  for k in range(BPW // L):
    sh = inv_v[pl.ds(k * L, L)]
    st = inv_v[pl.ds(BPW + k * L, L)]
    ixh[k // 4, pl.ds((k % 4) * L, L)] = sh
    ixt[k // 4, pl.ds((k % 4) * L, L)] = st

  for cp in rel_cps:
    cp.wait()

  NCH = BPW // 64

  def fire(ci):
    g = gbufs[ci % 2]
    return [
        pltpu.async_copy(oemb.at[ixh.at[ci]], g[0], sem),
        pltpu.async_copy(ore.at[ixh.at[ci]], g[1], sem),
        pltpu.async_copy(oim.at[ixh.at[ci]], g[2], sem),
        pltpu.async_copy(oemb.at[ixt.at[ci]], g[3], sem),
        pltpu.async_copy(ore.at[ixt.at[ci]], g[4], sem),
        pltpu.async_copy(oim.at[ixt.at[ci]], g[5], sem),
    ]

  accs = [jnp.zeros((L,), jnp.float32) for _ in range(9)]
  cps = fire(0)
  for ci in range(NCH):
    for cp in cps:
      cp.wait()
    if ci + 1 < NCH:
      cps = fire(ci + 1)
    g_eh, g_erh2, g_eih2, g_et, g_ert2, g_eit2 = gbufs[ci % 2]

    for gidx in range(64 // L):
      r0 = ci * 64 + gidx * L
      sl = pl.ds(r0, L)
      rid = idr_v[sl]
      rowi = jnp.arange(L, dtype=jnp.int32) + (gidx * L)
      res_acc = jnp.zeros((L,), jnp.float32)
      for d in range(HIDDEN):
        cd = jnp.full((L,), d, jnp.int32)
        erh = plsc.load_gather(g_erh2, [rowi, cd])
        eih = plsc.load_gather(g_eih2, [rowi, cd])
        eh = plsc.load_gather(g_eh, [rowi, cd])
        ert = plsc.load_gather(g_ert2, [rowi, cd])
        eit = plsc.load_gather(g_eit2, [rowi, cd])
        et = plsc.load_gather(g_et, [rowi, cd])
        rre = plsc.load_gather(rre_v, [rid, cd])
        rim = plsc.load_gather(rim_v, [rid, cd])
        r = plsc.load_gather(remb_v, [rid, cd])
        res_acc = res_acc + (rre * (erh * ert + eih * eit)
                             + rim * (erh * eit - eih * ert)
                             + eh * et * r)
        vals = (erh, eih, eh, ert, eit, et, rre, rim, r)
        for q in range(9):
          accs[q] = accs[q] + vals[q] * vals[q]
      res_v[pl.ds(r0, L)] = res_acc

  for q in range(9):
    reg_v[q, :] = accs[q]
  pltpu.sync_copy(res_v, res_hbm.at[pl.ds(base, BPW)])
  pltpu.sync_copy(reg_v, reg_hbm.at[wid])


@jax.jit
def _sc_calls(idh, idt, idr, emb, ere, eim, remb, rre, rim):
  mesh = plsc.VectorSubcoreMesh(core_axis_name="c", subcore_axis_name="s")
  p1 = pl.kernel(
      _p1_body,
      out_type=(
          jax.ShapeDtypeStruct((OUT_ROWS, 128), jnp.float32),
          jax.ShapeDtypeStruct((OUT_ROWS, 128), jnp.float32),
          jax.ShapeDtypeStruct((OUT_ROWS, 128), jnp.float32),
          jax.ShapeDtypeStruct((NW * CAP,), jnp.int32),
      ),
      mesh=mesh,
      scratch_types=[
          pltpu.VMEM((2048,), jnp.int32),
          pltpu.VMEM((CAP + L,), jnp.int32),
          pltpu.VMEM((CAP + L,), jnp.int32),
          pltpu.VMEM((CAP + L,), jnp.int32),
          pltpu.VMEM((CAP + L,), jnp.int32),
          pltpu.VMEM((272,), jnp.int32),
          pltpu.VMEM((272,), jnp.int32),
          pltpu.VMEM((2, K, L, 128), jnp.float32),
          pltpu.VMEM((2, K, L, 128), jnp.float32),
          pltpu.VMEM((2, K, L, 128), jnp.float32),
          pltpu.VMEM((8, 128), jnp.float32),
          pltpu.VMEM((8, 128), jnp.float32),
          pltpu.VMEM((8, 128), jnp.float32),
          pltpu.SemaphoreType.DMA,
          pltpu.SemaphoreType.DMA,
          pltpu.SemaphoreType.DMA,
      ],
      compiler_params=pltpu.CompilerParams(use_tc_tiling_on_sc=True,
                                           needs_layout_passes=False),
  )
  vemb, vre, vim, pos = p1(idh, idt, emb.T, ere.T, eim.T)

  p2 = pl.kernel(
      _p2_body,
      out_type=(
          jax.ShapeDtypeStruct((BATCH,), jnp.float32),
          jax.ShapeDtypeStruct((NW, 9, HIDDEN), jnp.float32),
      ),
      mesh=mesh,
      scratch_types=[
          pltpu.VMEM((NW * CAP,), jnp.int32),
          pltpu.VMEM((2 * BPW + L,), jnp.int32),
          pltpu.VMEM((BPW,), jnp.int32),
          pltpu.VMEM((BPW // 64, 64), jnp.int32),
          pltpu.VMEM((BPW // 64, 64), jnp.int32),
          [[pltpu.VMEM((64, HIDDEN), jnp.float32) for _ in range(6)]
           for _ in range(2)],
          pltpu.VMEM((REL_TOTAL, HIDDEN), jnp.float32),
          pltpu.VMEM((REL_TOTAL, HIDDEN), jnp.float32),
          pltpu.VMEM((REL_TOTAL, HIDDEN), jnp.float32),
          pltpu.VMEM((BPW,), jnp.float32),
          pltpu.VMEM((9, HIDDEN), jnp.float32),
          pltpu.SemaphoreType.DMA,
          pltpu.SemaphoreType.DMA,
      ],
      compiler_params=pltpu.CompilerParams(use_tc_tiling_on_sc=False,
                                           needs_layout_passes=False),
  )
  return p2(pos, idr, vemb.reshape(NW * CAP, HIDDEN),
            vre.reshape(NW * CAP, HIDDEN),
            vim.reshape(NW * CAP, HIDDEN), remb, rre, rim)


def _tc_body(res_ref, y_ref, reg_ref, out_ref):
  x = -(y_ref[...] * res_ref[...])
  sp = jnp.maximum(x, 0.0) + jnp.log(1.0 + jnp.exp(-jnp.abs(x)))
  loss = jnp.sum(sp) * (1.0 / BATCH)
  reg = reg_ref[...]
  scale = 1.0 / (BATCH * HIDDEN)
  m = [jnp.sum(reg[:, q * HIDDEN:(q + 1) * HIDDEN]) * scale for q in range(9)]
  regul = m[0] + m[1] * m[2] + m[3] + m[4] + m[5] + m[6] + m[7] + m[8]
  out_ref[...] = jnp.full((1, 1), loss + LMBDA * regul, jnp.float32)


@jax.jit
def _tc_call(res2, y2, reg2):
  return pl.pallas_call(
      _tc_body,
      out_shape=jax.ShapeDtypeStruct((1, 1), jnp.float32),
  )(res2, y2, reg2)


def kernel(id_h, id_t, id_r, y, ent_embeddings, ent_re, ent_im,
           rel_embeddings, rel_re, rel_im):
  idh = id_h.astype(jnp.int32)
  idt = id_t.astype(jnp.int32)
  idr = id_r.astype(jnp.int32)
  res, regp = _sc_calls(idh, idt, idr, ent_embeddings, ent_re, ent_im,
                        rel_embeddings, rel_re, rel_im)
  out = _tc_call(res.reshape(128, 128), y.reshape(128, 128),
                 regp.reshape(NW, 9 * HIDDEN))
  return out[0, 0]


# speedup vs baseline: 5.0905x; 1.0078x over previous
import functools

import jax
import jax.numpy as jnp
from jax import lax
from jax.experimental import pallas as pl
from jax.experimental.pallas import tpu as pltpu
from jax.experimental.pallas import tpu_sc as plsc

ENT_TOTAL = 1000000
REL_TOTAL = 1000
HIDDEN = 16
BATCH = 16384
LMBDA = 0.1

NC = 2
NS = 16
NW = NC * NS
BPW = BATCH // NW
L = 16

NBLK = (ENT_TOTAL + 127) // 128
LAST_BLK = NBLK - 1
CAP = 1280
ORPT = CAP // 8
OUT_ROWS = NW * ORPT
K = 8
NPH = 32
SENT = 2 * BATCH


def _iota():
  return lax.iota(jnp.int32, L)


def _p1_body(idh_hbm, idt_hbm, temb, tre, tim,
             oemb, ore, oim, opos,
             idstage, uns_id, uns_pos, srt_id, srt_pos, cnt_v, start_v,
             sl_emb, sl_re, sl_im, orow_emb, orow_re, orow_im,
             sem_a, sem_b, sem_f):
  wid = lax.axis_index("s") * NC + lax.axis_index("c")

  def scan_src(src_hbm, tag, off0):
    def chunk(c, off):
      pltpu.sync_copy(src_hbm.at[pl.ds(c * 2048, 2048)], idstage)

      def vec4(k4, off):
        vs, ms, cnts = [], [], []
        for u in range(4):
          v = idstage[pl.ds((k4 * 4 + u) * L, L)]
          m = ((v >> 7) & (NW - 1)) == wid
          vs.append(v)
          ms.append(m)
          cnts.append(plsc.all_reduce_population_count(m)[0])
        for u in range(4):
          plsc.store_compressed(uns_id.at[pl.ds(off, L)], vs[u], mask=ms[u])
          posv = (c * 2048 + tag + (k4 * 4 + u) * L) + _iota()
          plsc.store_compressed(uns_pos.at[pl.ds(off, L)], posv, mask=ms[u])
          off = off + cnts[u]
        return off

      return lax.fori_loop(0, 2048 // L // 4, vec4, off)

    return lax.fori_loop(0, BATCH // 2048, chunk, off0)

  n = scan_src(idh_hbm, 0, jnp.int32(0))
  n = scan_src(idt_hbm, BATCH, n)

  zero16 = jnp.zeros((L,), jnp.int32)
  for t in range(272 // L):
    cnt_v[pl.ds(t * L, L)] = zero16
    srt_pos[pl.ds(t * L, L)] = zero16 + SENT
  for t in range(272 // L, (CAP + L) // L):
    srt_pos[pl.ds(t * L, L)] = zero16 + SENT

  ones16 = jnp.ones((L,), jnp.int32)

  def count_vec(kv, _):
    mval = (kv * L + _iota()) < n
    j = uns_id[pl.ds(kv * L, L)] >> 12
    j = jnp.where(mval, j, 270)
    plsc.addupdate_scatter(cnt_v, [j], ones16, mask=mval)
    return 0

  lax.fori_loop(0, (CAP + L) // L, count_vec, 0)

  run = jnp.int32(0)
  for t in range(272 // L):
    v = cnt_v[pl.ds(t * L, L)]
    cs = plsc.cumsum(v)
    start_v[pl.ds(t * L, L)] = run + cs - v
    run = run + cs[L - 1]
  for t in range(272 // L):
    cnt_v[pl.ds(t * L, L)] = start_v[pl.ds(t * L, L)]

  def place_vec(kv, _):
    lanes = kv * L + _iota()
    mval = lanes < n
    idv = uns_id[pl.ds(kv * L, L)]
    posv = uns_pos[pl.ds(kv * L, L)]
    j = jnp.where(mval, idv >> 12, 270)
    cur = plsc.load_gather(cnt_v, [j])
    mi = mval.astype(jnp.int32)
    ordv = jnp.zeros((L,), jnp.int32)
    for l in range(L):
      same = (j == j[l]) & (_iota() > l)
      ordv = ordv + same.astype(jnp.int32) * mi[l]
    slot = cur + ordv
    plsc.store_scatter(srt_id, [slot], idv, mask=mval)
    plsc.store_scatter(srt_pos, [slot], posv, mask=mval)
    plsc.addupdate_scatter(cnt_v, [j], ones16, mask=mval)
    return 0

  lax.fori_loop(0, (CAP + L) // L, place_vec, 0)

  tables = ((temb, sl_emb, orow_emb, oemb),
            (tre, sl_re, orow_re, ore),
            (tim, sl_im, orow_im, oim))

  def dma_phase(ph, g, fire):
    sem = sem_a if g == 0 else sem_b
    for b in range(K):
      j = ph * K + b
      blk = wid + NW * j
      col = pl.multiple_of(jnp.where(blk <= LAST_BLK, blk, 0) * 128, 128)
      for tbl, sl, _o, _oh in tables:
        cp = pltpu.make_async_copy(tbl.at[:, pl.ds(col, 128)],
                                   sl.at[g].at[b], sem)
        if fire:
          cp.start()
        else:
          cp.wait()

  def process_phase(ph, g):
    for b in range(K):
      j = ph * K + b
      lo = start_v[pl.ds(j, L)][0]
      hi = start_v[pl.ds(j + 1, L)][0]

      def entry(e, _):
        idv = srt_id[pl.ds(e, L)]
        loff = idv[0] & 127
        coli = jnp.full((L,), loff, jnp.int32)
        orow = (e >> 3) & 7
        osl = pl.ds((e & 7) * HIDDEN, HIDDEN)
        for _tbl, sl, orow_v, _oh in tables:
          v = plsc.load_gather(sl.at[g].at[b], [_iota(), coli])
          orow_v[orow, osl] = v

        @pl.when((e & 63) == 63)
        def _():
          rb = pl.multiple_of(wid * ORPT + ((e >> 6) << 3), 8)
          fcps = [pltpu.async_copy(orow_v, out_hbm.at[pl.ds(rb, 8)], sem_f)
                  for _tbl, _sl, orow_v, out_hbm in tables]
          for fcp in fcps:
            fcp.wait()

        return 0

      lax.fori_loop(lo, hi, entry, 0)

  dma_phase(0, 0, True)

  def pair(p, _):
    dma_phase(2 * p + 1, 1, True)
    dma_phase(2 * p, 0, False)
    process_phase(2 * p, 0)

    @pl.when(p < NPH // 2 - 1)
    def _():
      dma_phase(2 * p + 2, 0, True)

    dma_phase(2 * p + 1, 1, False)
    process_phase(2 * p + 1, 1)
    return 0

  lax.fori_loop(0, NPH // 2, pair, 0)

  @pl.when(n > 0)
  def _():
    rb = pl.multiple_of(wid * ORPT + (((n - 1) >> 6) << 3), 8)
    fcps = [pltpu.async_copy(orow_v, out_hbm.at[pl.ds(rb, 8)], sem_f)
            for _tbl, _sl, orow_v, out_hbm in tables]
    for fcp in fcps:
      fcp.wait()

  pltpu.sync_copy(srt_pos.at[pl.ds(0, CAP)], opos.at[pl.ds(wid * CAP, CAP)])


def _p2_body(opos_hbm, idr_hbm, oemb, ore, oim, remb_hbm, rre_hbm, rim_hbm,
             res_hbm, reg_hbm,
             posall, inv_v, idr_v, ixh, ixt,
             gbufs, remb_v, rre_v, rim_v, res_v, reg_v, sem, semr):
  wid = lax.axis_index("s") * NC + lax.axis_index("c")
  base = wid * BPW

  pltpu.sync_copy(opos_hbm, posall)
  pltpu.sync_copy(idr_hbm.at[pl.ds(base, BPW)], idr_v)
  rel_cps = [pltpu.async_copy(remb_hbm, remb_v, semr),
             pltpu.async_copy(rre_hbm, rre_v, semr),
             pltpu.async_copy(rim_hbm, rim_v, semr)]

  def vec4(k4, _):
    for u in range(4):
      kv = k4 * 4 + u
      posv = posall[pl.ds(kv * L, L)]
      slotv = kv * L + _iota()
      in_h = (posv >= base) & (posv < base + BPW)
      in_t = (posv >= BATCH + base) & (posv < BATCH + base + BPW)
      m = in_h | in_t
      idx = jnp.where(in_h, posv - base, posv - (BATCH + base) + BPW)
      idx = jnp.where(m, idx, 2 * BPW)
      plsc.store_scatter(inv_v, [idx], slotv, mask=m)
    return 0

  lax.fori_loop(0, (NW * CAP) // L // 4, vec4, 0)

  for k in range(BPW // L):
    sh = inv_v[pl.ds(k * L, L)]
    st = inv_v[pl.ds(BPW + k * L, L)]
    ixh[k // 4, pl.ds((k % 4) * L, L)] = sh
    ixt[k // 4, pl.ds((k % 4) * L, L)] = st

  for cp in rel_cps:
    cp.wait()

  NCH = BPW // 64

  def fire(ci):
    g = gbufs[ci % 2]
    return [
        pltpu.async_copy(oemb.at[ixh.at[ci]], g[0], sem),
        pltpu.async_copy(ore.at[ixh.at[ci]], g[1], sem),
        pltpu.async_copy(oim.at[ixh.at[ci]], g[2], sem),
        pltpu.async_copy(oemb.at[ixt.at[ci]], g[3], sem),
        pltpu.async_copy(ore.at[ixt.at[ci]], g[4], sem),
        pltpu.async_copy(oim.at[ixt.at[ci]], g[5], sem),
    ]

  accs = [jnp.zeros((L,), jnp.float32) for _ in range(9)]
  cps = fire(0)
  for ci in range(NCH):
    for cp in cps:
      cp.wait()
    if ci + 1 < NCH:
      cps = fire(ci + 1)
    g_eh, g_erh2, g_eih2, g_et, g_ert2, g_eit2 = gbufs[ci % 2]

    for gidx in range(64 // L):
      r0 = ci * 64 + gidx * L
      sl = pl.ds(r0, L)
      rid = idr_v[sl]
      rowi = jnp.arange(L, dtype=jnp.int32) + (gidx * L)
      res_acc = jnp.zeros((L,), jnp.float32)
      for d in range(HIDDEN):
        cd = jnp.full((L,), d, jnp.int32)
        erh = plsc.load_gather(g_erh2, [rowi, cd])
        eih = plsc.load_gather(g_eih2, [rowi, cd])
        eh = plsc.load_gather(g_eh, [rowi, cd])
        ert = plsc.load_gather(g_ert2, [rowi, cd])
        eit = plsc.load_gather(g_eit2, [rowi, cd])
        et = plsc.load_gather(g_et, [rowi, cd])
        rre = plsc.load_gather(rre_v, [rid, cd])
        rim = plsc.load_gather(rim_v, [rid, cd])
        r = plsc.load_gather(remb_v, [rid, cd])
        res_acc = res_acc + (rre * (erh * ert + eih * eit)
                             + rim * (erh * eit - eih * ert)
                             + eh * et * r)
        vals = (erh, eih, eh, ert, eit, et, rre, rim, r)
        for q in range(9):
          accs[q] = accs[q] + vals[q] * vals[q]
      res_v[pl.ds(r0, L)] = res_acc

  for q in range(9):
    reg_v[q, :] = accs[q]
  pltpu.sync_copy(res_v, res_hbm.at[pl.ds(base, BPW)])
  pltpu.sync_copy(reg_v, reg_hbm.at[wid])


@jax.jit
def _sc_calls(idh, idt, idr, emb, ere, eim, remb, rre, rim):
  mesh = plsc.VectorSubcoreMesh(core_axis_name="c", subcore_axis_name="s")
  p1 = pl.kernel(
      _p1_body,
      out_type=(
          jax.ShapeDtypeStruct((OUT_ROWS, 128), jnp.float32),
          jax.ShapeDtypeStruct((OUT_ROWS, 128), jnp.float32),
          jax.ShapeDtypeStruct((OUT_ROWS, 128), jnp.float32),
          jax.ShapeDtypeStruct((NW * CAP,), jnp.int32),
      ),
      mesh=mesh,
      scratch_types=[
          pltpu.VMEM((2048,), jnp.int32),
          pltpu.VMEM((CAP + L,), jnp.int32),
          pltpu.VMEM((CAP + L,), jnp.int32),
          pltpu.VMEM((CAP + L,), jnp.int32),
          pltpu.VMEM((CAP + L,), jnp.int32),
          pltpu.VMEM((272,), jnp.int32),
          pltpu.VMEM((272,), jnp.int32),
          pltpu.VMEM((2, K, L, 128), jnp.float32),
          pltpu.VMEM((2, K, L, 128), jnp.float32),
          pltpu.VMEM((2, K, L, 128), jnp.float32),
          pltpu.VMEM((8, 128), jnp.float32),
          pltpu.VMEM((8, 128), jnp.float32),
          pltpu.VMEM((8, 128), jnp.float32),
          pltpu.SemaphoreType.DMA,
          pltpu.SemaphoreType.DMA,
          pltpu.SemaphoreType.DMA,
      ],
      compiler_params=pltpu.CompilerParams(use_tc_tiling_on_sc=True,
                                           needs_layout_passes=False),
  )
  vemb, vre, vim, pos = p1(idh, idt, emb.T, ere.T, eim.T)

  p2 = pl.kernel(
      _p2_body,
      out_type=(
          jax.ShapeDtypeStruct((BATCH,), jnp.float32),
          jax.ShapeDtypeStruct((NW, 9, HIDDEN), jnp.float32),
      ),
      mesh=mesh,
      scratch_types=[
          pltpu.VMEM((NW * CAP,), jnp.int32),
          pltpu.VMEM((2 * BPW + L,), jnp.int32),
          pltpu.VMEM((BPW,), jnp.int32),
          pltpu.VMEM((BPW // 64, 64), jnp.int32),
          pltpu.VMEM((BPW // 64, 64), jnp.int32),
          [[pltpu.VMEM((64, HIDDEN), jnp.float32) for _ in range(6)]
           for _ in range(2)],
          pltpu.VMEM((REL_TOTAL, HIDDEN), jnp.float32),
          pltpu.VMEM((REL_TOTAL, HIDDEN), jnp.float32),
          pltpu.VMEM((REL_TOTAL, HIDDEN), jnp.float32),
          pltpu.VMEM((BPW,), jnp.float32),
          pltpu.VMEM((9, HIDDEN), jnp.float32),
          pltpu.SemaphoreType.DMA,
          pltpu.SemaphoreType.DMA,
      ],
      compiler_params=pltpu.CompilerParams(use_tc_tiling_on_sc=False,
                                           needs_layout_passes=False),
  )
  return p2(pos, idr, vemb.reshape(NW * CAP, HIDDEN),
            vre.reshape(NW * CAP, HIDDEN),
            vim.reshape(NW * CAP, HIDDEN), remb, rre, rim)


def _tc_body(res_ref, y_ref, reg_ref, out_ref):
  x = -(y_ref[...] * res_ref[...])
  sp = jnp.maximum(x, 0.0) + jnp.log(1.0 + jnp.exp(-jnp.abs(x)))
  loss = jnp.sum(sp) * (1.0 / BATCH)
  reg = reg_ref[...]
  scale = 1.0 / (BATCH * HIDDEN)
  m = [jnp.sum(reg[:, q * HIDDEN:(q + 1) * HIDDEN]) * scale for q in range(9)]
  regul = m[0] + m[1] * m[2] + m[3] + m[4] + m[5] + m[6] + m[7] + m[8]
  out_ref[...] = jnp.full((1, 1), loss + LMBDA * regul, jnp.float32)


@jax.jit
def _tc_call(res2, y2, reg2):
  return pl.pallas_call(
      _tc_body,
      out_shape=jax.ShapeDtypeStruct((1, 1), jnp.float32),
  )(res2, y2, reg2)


def kernel(id_h, id_t, id_r, y, ent_embeddings, ent_re, ent_im,
           rel_embeddings, rel_re, rel_im):
  idh = id_h.astype(jnp.int32)
  idt = id_t.astype(jnp.int32)
  idr = id_r.astype(jnp.int32)
  res, regp = _sc_calls(idh, idt, idr, ent_embeddings, ent_re, ent_im,
                        rel_embeddings, rel_re, rel_im)
  out = _tc_call(res.reshape(128, 128), y.reshape(128, 128),
                 regp.reshape(NW, 9 * HIDDEN))
  return out[0, 0]
